# baseline (speedup 1.0000x reference)
import functools

import jax
import jax.numpy as jnp
from jax import lax
from jax.experimental import pallas as pl
from jax.experimental.pallas import tpu as pltpu
from jax.experimental.pallas import tpu_sc as plsc

S = 50
NB = 16384
DIM = 64
PDIM = 128
W = 32
BC = 128
BPW = NB // W
CPB = BPW // BC
NBLK = S * CPB
NBUF = 3
NTB = 2


def _embedding_lookup(xg, tpad):
    mesh = plsc.VectorSubcoreMesh(core_axis_name="c", subcore_axis_name="s")

    @functools.partial(
        pl.kernel,
        out_type=jax.ShapeDtypeStruct((S, DIM, NB), jnp.float32),
        mesh=mesh,
        scratch_types=[
            pltpu.VMEM((NBLK, BC), jnp.int32),
            pltpu.VMEM((NBUF, BC, PDIM), jnp.float32),
            pltpu.VMEM((NTB, DIM, BC), jnp.float32),
            pltpu.SemaphoreType.DMA((NBUF,)),
            pltpu.SemaphoreType.DMA((NTB,)),
        ],
        compiler_params=pltpu.CompilerParams(use_tc_tiling_on_sc=True,
                                             needs_layout_passes=False),
    )
    def body(xg_hbm, tpad_hbm, out_hbm, idx_v, rows_v, trb_v, g_sem, o_sem):
        wid = lax.axis_index("s") * 2 + lax.axis_index("c")
        b_base = wid * BPW
        pltpu.sync_copy(xg_hbm.at[wid], idx_v)

        def start_gather(j):
            buf = lax.rem(j, NBUF)
            pltpu.async_copy(tpad_hbm.at[idx_v.at[j]], rows_v.at[buf],
                             g_sem.at[buf])

        def wait_gather(j):
            buf = lax.rem(j, NBUF)
            pltpu.make_async_copy(tpad_hbm.at[idx_v.at[j]], rows_v.at[buf],
                                  g_sem.at[buf]).wait()

        def out_slices(j, tb):
            s = lax.div(j, CPB)
            c = lax.rem(j, CPB)
            return trb_v.at[tb], out_hbm.at[s, :, pl.ds(b_base + c * BC, BC)]

        def start_out(j, tb):
            src, dst = out_slices(j, tb)
            pltpu.async_copy(src, dst, o_sem.at[tb])

        def wait_out(j, tb):
            src, dst = out_slices(j, tb)
            pltpu.make_async_copy(src, dst, o_sem.at[tb]).wait()

        for j in range(NBUF):
            start_gather(j)

        iota16 = lax.iota(jnp.int32, 16)

        def transpose_block(buf, tb):
            def fbody(f, carry):
                col = jnp.full((16,), f, jnp.int32)
                for g in range(BC // 16):
                    bvec = iota16 + (g * 16)
                    v = plsc.load_gather(rows_v.at[buf], [bvec, col])
                    trb_v[tb, f, pl.ds(g * 16, 16)] = v
                return carry

            lax.fori_loop(0, DIM, fbody, 0)

        def step(j, carry):
            buf = lax.rem(j, NBUF)
            tb = lax.rem(j, NTB)
            wait_gather(j)

            @pl.when(j >= NTB)
            def _():
                wait_out(j - NTB, tb)

            transpose_block(buf, tb)
            start_out(j, tb)

            @pl.when(j + NBUF < NBLK)
            def _():
                start_gather(j + NBUF)

            return carry

        lax.fori_loop(0, NBLK, step, 0)
        for k in range(NTB):
            j = NBLK - NTB + k
            wait_out(j, lax.rem(j, NTB))

    return body(xg, tpad)


def kernel(x, table):
    xg = (x.T.astype(jnp.int32)
          .reshape(S, W, CPB, BC)
          .transpose(1, 0, 2, 3)
          .reshape(W, NBLK, BC))
    tpad = jnp.pad(table, ((0, 0), (0, PDIM - DIM)))
    out_t = _embedding_lookup(xg, tpad)
    return out_t.transpose(2, 0, 1)


# speedup vs baseline: 1.4069x; 1.4069x over previous
import functools

import jax
import jax.numpy as jnp
from jax import lax
from jax.experimental import pallas as pl
from jax.experimental.pallas import tpu as pltpu
from jax.experimental.pallas import tpu_sc as plsc

S = 50
NB = 16384
DIM = 64
W = 32
CHUNK = 128
NCH = NB * S // (W * CHUNK)
NBUF = 4
INFLIGHT = 3


def _embedding_lookup(xg, table):
    mesh = plsc.VectorSubcoreMesh(core_axis_name="c", subcore_axis_name="s")

    @functools.partial(
        pl.kernel,
        out_type=jax.ShapeDtypeStruct((W, NCH, CHUNK, DIM), jnp.float32),
        mesh=mesh,
        scratch_types=[
            pltpu.VMEM((NCH, CHUNK), jnp.int32),
            pltpu.VMEM((NBUF, CHUNK, DIM), jnp.float32),
            pltpu.SemaphoreType.DMA((NBUF,)),
            pltpu.SemaphoreType.DMA((NBUF,)),
        ],
        compiler_params=pltpu.CompilerParams(use_tc_tiling_on_sc=False),
    )
    def body(xg_hbm, table_hbm, out_hbm, idx_v, rows_v, g_sem, o_sem):
        wid = lax.axis_index("s") * 2 + lax.axis_index("c")
        pltpu.sync_copy(xg_hbm.at[wid], idx_v)

        def start_gather(j):
            buf = lax.rem(j, NBUF)
            pltpu.async_copy(table_hbm.at[idx_v.at[j]], rows_v.at[buf],
                             g_sem.at[buf])

        def wait_gather(j):
            buf = lax.rem(j, NBUF)
            pltpu.make_async_copy(table_hbm.at[idx_v.at[j]], rows_v.at[buf],
                                  g_sem.at[buf]).wait()

        def start_out(j):
            buf = lax.rem(j, NBUF)
            pltpu.async_copy(rows_v.at[buf], out_hbm.at[wid, j],
                             o_sem.at[buf])

        def wait_out(j):
            buf = lax.rem(j, NBUF)
            pltpu.make_async_copy(rows_v.at[buf], out_hbm.at[wid, j],
                                  o_sem.at[buf]).wait()

        for j in range(INFLIGHT):
            start_gather(j)

        def step(j, carry):
            wait_gather(j)
            start_out(j)

            @pl.when(j + INFLIGHT < NCH)
            def _():
                @pl.when(j >= 1)
                def _():
                    wait_out(j - 1)

                start_gather(j + INFLIGHT)

            return carry

        lax.fori_loop(0, NCH, step, 0)
        for j in range(NCH - NBUF, NCH):
            wait_out(j)

    return body(xg, table)


def kernel(x, table):
    xg = x.astype(jnp.int32).reshape(W, NCH, CHUNK)
    out = _embedding_lookup(xg, table)
    return out.reshape(NB, S, DIM)


# speedup vs baseline: 1.4681x; 1.0435x over previous
import functools

import jax
import jax.numpy as jnp
from jax import lax
from jax.experimental import pallas as pl
from jax.experimental.pallas import tpu as pltpu
from jax.experimental.pallas import tpu_sc as plsc

S = 50
NB = 16384
DIM = 64
W = 32
CHUNK = 128
CPS = NB // CHUNK
NCH = NB * S // (W * CHUNK)
NBUF = 4
INFLIGHT = 3


def _embedding_lookup(xg, table):
    mesh = plsc.VectorSubcoreMesh(core_axis_name="c", subcore_axis_name="s")

    @functools.partial(
        pl.kernel,
        out_type=jax.ShapeDtypeStruct((S, NB, DIM), jnp.float32),
        mesh=mesh,
        scratch_types=[
            pltpu.VMEM((NCH, CHUNK), jnp.int32),
            pltpu.VMEM((NBUF, CHUNK, DIM), jnp.float32),
            pltpu.SemaphoreType.DMA((NBUF,)),
            pltpu.SemaphoreType.DMA((NBUF,)),
        ],
        compiler_params=pltpu.CompilerParams(use_tc_tiling_on_sc=False),
    )
    def body(xg_hbm, table_hbm, out_hbm, idx_v, rows_v, g_sem, o_sem):
        wid = lax.axis_index("s") * 2 + lax.axis_index("c")
        pltpu.sync_copy(xg_hbm.at[wid], idx_v)

        def start_gather(j):
            buf = lax.rem(j, NBUF)
            pltpu.async_copy(table_hbm.at[idx_v.at[j]], rows_v.at[buf],
                             g_sem.at[buf])

        def wait_gather(j):
            buf = lax.rem(j, NBUF)
            pltpu.make_async_copy(table_hbm.at[idx_v.at[j]], rows_v.at[buf],
                                  g_sem.at[buf]).wait()

        def out_slice(j):
            g = wid * NCH + j
            s = lax.div(g, CPS)
            c = lax.rem(g, CPS)
            return out_hbm.at[s, pl.ds(c * CHUNK, CHUNK)]

        def start_out(j):
            buf = lax.rem(j, NBUF)
            pltpu.async_copy(rows_v.at[buf], out_slice(j), o_sem.at[buf])

        def wait_out(j):
            buf = lax.rem(j, NBUF)
            pltpu.make_async_copy(rows_v.at[buf], out_slice(j),
                                  o_sem.at[buf]).wait()

        for j in range(INFLIGHT):
            start_gather(j)

        def step(j, carry):
            wait_gather(j)
            start_out(j)

            @pl.when(j + INFLIGHT < NCH)
            def _():
                @pl.when(j >= 1)
                def _():
                    wait_out(j - 1)

                start_gather(j + INFLIGHT)

            return carry

        lax.fori_loop(0, NCH, step, 0)
        for j in range(NCH - NBUF, NCH):
            wait_out(j)

    return body(xg, table)


def kernel(x, table):
    xg = x.T.astype(jnp.int32).reshape(W, NCH, CHUNK)
    out = _embedding_lookup(xg, table)
    return out.transpose(1, 0, 2)
